# baseline (speedup 1.0000x reference)
import jax
import jax.numpy as jnp
from jax import lax
from jax.experimental import pallas as pl
from jax.experimental.pallas import tpu as pltpu
from jax.experimental.pallas import tpu_sc as plsc

VOCAB = 1000000
EMBED_DIM = 64
BATCH = 4096
SEQ_LEN = 200

NC = 2
NS = 16
NW = NC * NS

NSPLIT = 2
SEQ_H = SEQ_LEN // NSPLIT
N = BATCH * SEQ_H
NB = N // NW
C = 256
GC = NB // C
K = 3
KP = 2
NBUF = K + KP
assert (GC - K - KP) % NBUF == 0


def _emb_body(table_hbm, text_hbm, out_hbm, idx_v, *rest):
    rows = rest[:NBUF]
    gsems = rest[NBUF:2 * NBUF]
    wsems = rest[2 * NBUF:]

    wid = lax.axis_index("s") * NC + lax.axis_index("c")

    pltpu.sync_copy(text_hbm.at[wid], idx_v)

    def idx_slice(g):
        return idx_v.at[pl.ds(g * C, C)]

    def start_gather(g, b):
        pltpu.async_copy(table_hbm.at[idx_slice(g)], rows[b], gsems[b])

    def wait_gather(g, b):
        pltpu.make_async_copy(table_hbm.at[idx_slice(g)], rows[b],
                              gsems[b]).wait()

    def start_wb(g, b):
        pltpu.async_copy(rows[b], out_hbm.at[wid, g], wsems[b])

    def wait_wb(g, b):
        pltpu.make_async_copy(rows[b], out_hbm.at[wid, g], wsems[b]).wait()

    for g in range(K):
        start_gather(g, g % NBUF)
    for g in range(KP):
        wait_gather(g, g % NBUF)
        start_wb(g, g % NBUF)
        start_gather(g + K, (g + K) % NBUF)

    def group(j, _):
        for t in range(NBUF):
            g = KP + j * NBUF + t
            b = (KP + t) % NBUF
            wait_gather(g, b)
            start_wb(g, b)
            wait_wb(g - KP, (b - KP) % NBUF)
            start_gather(g + K, (b + K) % NBUF)
        return 0

    lax.fori_loop(0, (GC - K - KP) // NBUF, group, 0, unroll=False)

    for g in range(GC - K, GC):
        b = g % NBUF
        wait_gather(g, b)
        start_wb(g, b)
        wait_wb(g - KP, (g - KP) % NBUF)
    for g in range(GC - KP, GC):
        wait_wb(g, g % NBUF)


@jax.jit
def _embed(text_flat, table):
    mesh = plsc.VectorSubcoreMesh(core_axis_name="c", subcore_axis_name="s")
    k = pl.kernel(
        _emb_body,
        out_type=jax.ShapeDtypeStruct((NW, GC, C, EMBED_DIM), jnp.float32),
        mesh=mesh,
        scratch_types=(
            [pltpu.VMEM((NB,), jnp.int32)]
            + [pltpu.VMEM((C, EMBED_DIM), jnp.float32) for _ in range(NBUF)]
            + [pltpu.SemaphoreType.DMA for _ in range(2 * NBUF)]
        ),
        compiler_params=pltpu.CompilerParams(
            use_tc_tiling_on_sc=False,
            skip_device_barrier=True,
        ),
    )
    return k(table, text_flat)


def kernel(text, table):
    halves = []
    for h in range(NSPLIT):
        th = text[:, h * SEQ_H:(h + 1) * SEQ_H].reshape(NW, NB)
        oh = _embed(th, table)
        halves.append(oh.reshape(BATCH, SEQ_H, EMBED_DIM))
    return jnp.concatenate(halves, axis=1)


# speedup vs baseline: 1.5821x; 1.5821x over previous
import jax
import jax.numpy as jnp
from jax import lax
from jax.experimental import pallas as pl
from jax.experimental.pallas import tpu as pltpu
from jax.experimental.pallas import tpu_sc as plsc

VOCAB = 1000000
EMBED_DIM = 64
BATCH = 4096
SEQ_LEN = 200

NC = 2
NS = 16
NW = NC * NS

N = BATCH * SEQ_LEN
NB = N // NW
C = 256
GC = NB // C
K = 3
KP = 2
NBUF = K + KP
assert (GC - K - KP) % NBUF == 0


def _emb_body(table_hbm, text_hbm, out_hbm, idx_v, *rest):
    rows = rest[:NBUF]
    gsems = rest[NBUF:2 * NBUF]
    wsems = rest[2 * NBUF:]

    wid = lax.axis_index("s") * NC + lax.axis_index("c")

    pltpu.sync_copy(text_hbm.at[wid], idx_v)

    def idx_slice(g):
        return idx_v.at[pl.ds(g * C, C)]

    def start_gather(g, b):
        pltpu.async_copy(table_hbm.at[idx_slice(g)], rows[b], gsems[b])

    def wait_gather(g, b):
        pltpu.make_async_copy(table_hbm.at[idx_slice(g)], rows[b],
                              gsems[b]).wait()

    def start_wb(g, b):
        pltpu.async_copy(rows[b], out_hbm.at[wid, g], wsems[b])

    def wait_wb(g, b):
        pltpu.make_async_copy(rows[b], out_hbm.at[wid, g], wsems[b]).wait()

    for g in range(K):
        start_gather(g, g % NBUF)
    for g in range(KP):
        wait_gather(g, g % NBUF)
        start_wb(g, g % NBUF)
        start_gather(g + K, (g + K) % NBUF)

    def group(j, _):
        for t in range(NBUF):
            g = KP + j * NBUF + t
            b = (KP + t) % NBUF
            wait_gather(g, b)
            start_wb(g, b)
            wait_wb(g - KP, (b - KP) % NBUF)
            start_gather(g + K, (b + K) % NBUF)
        return 0

    lax.fori_loop(0, (GC - K - KP) // NBUF, group, 0, unroll=False)

    for g in range(GC - K, GC):
        b = g % NBUF
        wait_gather(g, b)
        start_wb(g, b)
        wait_wb(g - KP, (g - KP) % NBUF)
    for g in range(GC - KP, GC):
        wait_wb(g, g % NBUF)


@jax.jit
def _embed(text_flat, table):
    mesh = plsc.VectorSubcoreMesh(core_axis_name="c", subcore_axis_name="s")
    k = pl.kernel(
        _emb_body,
        out_type=jax.ShapeDtypeStruct((NW, GC, C, EMBED_DIM), jnp.float32),
        mesh=mesh,
        scratch_types=(
            [pltpu.VMEM((NB,), jnp.int32)]
            + [pltpu.VMEM((C, EMBED_DIM), jnp.float32) for _ in range(NBUF)]
            + [pltpu.SemaphoreType.DMA for _ in range(2 * NBUF)]
        ),
        compiler_params=pltpu.CompilerParams(
            use_tc_tiling_on_sc=False,
            skip_device_barrier=True,
        ),
    )
    return k(table, text_flat)


def kernel(text, table):
    text_flat = text.reshape(NW, NB)
    out = _embed(text_flat, table)
    return out.reshape(BATCH, SEQ_LEN, EMBED_DIM)
